# baseline (speedup 1.0000x reference)
import jax
import jax.numpy as jnp
from jax import lax
from jax.experimental import pallas as pl
from jax.experimental.pallas import tpu as pltpu
from jax.experimental.pallas import tpu_sc as plsc

_DIM = 100
_NS = 144
_B = 32
_V = 16
_F32 = jnp.float32


def _sc_body(coords_hbm, mask_hbm, out_hbm, maskv, pv, outv, sem):
    b = lax.axis_index("s") * 2 + lax.axis_index("c")

    mask_cp = pltpu.async_copy(mask_hbm.at[b], maskv, sem)
    pltpu.sync_copy(coords_hbm.at[b], pv)

    def shoelace(ref, k):
        x = ref[pl.ds(k * 64 + 0, _V)]
        y = ref[pl.ds(k * 64 + 16, _V)]
        xn = ref[pl.ds(k * 64 + 32, _V)]
        yn = ref[pl.ds(k * 64 + 48, _V)]
        ymax = jnp.max(y)
        s = jnp.sum((xn - x) * (ymax - (yn + y) * 0.5))
        return jnp.abs(s), x, y, xn, yn

    pred_area, px, py, pxn, pyn = shoelace(pv, 0)
    gt_area, _, _, _, _ = shoelace(pv, 1)

    xmin = jnp.minimum(px, pxn) - 0.001
    xmax = jnp.maximum(px, pxn) + 0.001
    ymin = jnp.minimum(py, pyn) - 0.001
    ymax_e = jnp.maximum(py, pyn) + 0.001
    sign = jnp.where(pxn > px, 1.0, -1.0).astype(_F32)

    def unit_vec(sx, sy, ex, ey):
        vxr = ex - sx + 1e-6
        vyr = ey - sy + 1e-6
        nsq = vxr * vxr + vyr * vyr
        yi = 0x5F3759DF - (plsc.bitcast(nsq, jnp.int32) >> 1)
        ry = plsc.bitcast(yi, _F32)
        for _ in range(3):
            ry = ry * (1.5 - 0.5 * nsq * ry * ry)
        return vxr * ry, vyr * ry, nsq * ry

    fvx, fvy, fnorm = unit_vec(px, py, pxn, pyn)
    bvx, bvy, bnorm = unit_vec(pxn, pyn, px, py)

    def sample(dsf, sx, sy, vx, vy, flp):
        xs = sx + dsf * vx
        ys = sy + dsf * vy
        keep = ((xs <= xmax) & (xs >= xmin)
                & (ys <= ymax_e) & (ys >= ymin))
        fl = xs.astype(jnp.int32)
        uniq = keep & (fl != flp)
        y0i = ys.astype(jnp.int32)
        fx = xs - fl.astype(_F32)
        fy = ys - y0i.astype(_F32)
        y1i = y0i + 1
        k0 = (y0i >> 2) * _DIM + fl
        k1 = (y1i >> 2) * _DIM + fl
        b0 = plsc.load_gather(maskv, [k0], mask=uniq) >> ((y0i & 3) << 3)
        b1 = plsc.load_gather(maskv, [k1], mask=uniq) >> ((y1i & 3) << 3)
        m00 = (b0 & 1).astype(_F32)
        m10 = ((b0 >> 1) & 1).astype(_F32)
        m01 = (b1 & 1).astype(_F32)
        m11 = ((b1 >> 1) & 1).astype(_F32)
        a = m00 + fy * (m01 - m00)
        bb = m10 + fy * (m11 - m10)
        v = a + fx * (bb - a)
        return jnp.where(uniq, v, 0.0), fl

    def step(j, carry):
        acc_f, acc_b, flp_f, flp_b = carry
        dsf = (j * 2).astype(_F32)
        vf0, fl_f0 = sample(dsf, px, py, fvx, fvy, flp_f)
        vb0, fl_b0 = sample(dsf, pxn, pyn, bvx, bvy, flp_b)
        vf1, fl_f1 = sample(dsf + 1.0, px, py, fvx, fvy, fl_f0)
        vb1, fl_b1 = sample(dsf + 1.0, pxn, pyn, bvx, bvy, fl_b0)
        return acc_f + (vf0 + vf1), acc_b + (vb0 + vb1), fl_f1, fl_b1

    trip = jnp.minimum(
        jnp.max(jnp.maximum(fnorm, bnorm).astype(jnp.int32)) + 3, _NS)
    zerov = jnp.zeros_like(px)
    sentinel = jnp.zeros((_V,), jnp.int32) - (1 << 20)
    mask_cp.wait()
    acc_f, acc_b, _, _ = lax.fori_loop(
        0, (trip + 1) >> 1, step, (zerov, zerov, sentinel, sentinel))
    int_area = jnp.abs(jnp.sum(sign * (acc_f + acc_b) * 0.5))
    union = pred_area + gt_area - int_area
    zeros = jnp.zeros((_V,), _F32)
    outv[...] = (zeros + int_area) / (zeros + union)
    pltpu.sync_copy(outv, out_hbm.at[b])


@jax.jit
def kernel(poly, gt, gt_mask):
    def rows(p):
        x = p[:, :, 0]
        y = p[:, :, 1]
        return [x, y, jnp.roll(x, -1, axis=1), jnp.roll(y, -1, axis=1)]

    coords = jnp.concatenate(rows(poly) + rows(gt), axis=1)
    mi = gt_mask.astype(jnp.int32).reshape(_B, _DIM, _DIM)
    mr = jnp.pad(mi[:, :, 1:], ((0, 0), (0, 0), (0, 1)))
    q = mi | (mr << 1)
    w = (q[:, 0::4] | (q[:, 1::4] << 8)
         | (q[:, 2::4] << 16) | (q[:, 3::4] << 24))
    maskf = jnp.pad(w.reshape(_B, (_DIM * _DIM) // 4), ((0, 0), (0, 60)))
    mesh = plsc.VectorSubcoreMesh(core_axis_name="c", subcore_axis_name="s")
    out = pl.kernel(
        _sc_body,
        mesh=mesh,
        compiler_params=pltpu.CompilerParams(
            needs_layout_passes=False, use_tc_tiling_on_sc=False),
        out_type=jax.ShapeDtypeStruct((_B, _V), _F32),
        scratch_types=[
            pltpu.VMEM((2560,), jnp.int32),
            pltpu.VMEM((8 * _V,), _F32),
            pltpu.VMEM((_V,), _F32),
            pltpu.SemaphoreType.DMA,
        ],
    )(coords, maskf)
    return out[:, 0]


# speedup vs baseline: 1.0558x; 1.0558x over previous
import jax
import jax.numpy as jnp
from jax import lax
from jax.experimental import pallas as pl
from jax.experimental.pallas import tpu as pltpu
from jax.experimental.pallas import tpu_sc as plsc

_DIM = 100
_NS = 144
_B = 32
_V = 16
_F32 = jnp.float32


def _sc_body(coords_hbm, mask_hbm, out_hbm, maskv, pv, outv, sem):
    b = lax.axis_index("s") * 2 + lax.axis_index("c")

    mask_cp = pltpu.async_copy(mask_hbm.at[b], maskv, sem)
    pltpu.sync_copy(coords_hbm.at[b], pv)

    def shoelace(ref, k):
        x = ref[pl.ds(k * 64 + 0, _V)]
        y = ref[pl.ds(k * 64 + 16, _V)]
        xn = ref[pl.ds(k * 64 + 32, _V)]
        yn = ref[pl.ds(k * 64 + 48, _V)]
        ymax = jnp.max(y)
        s = jnp.sum((xn - x) * (ymax - (yn + y) * 0.5))
        return jnp.abs(s), x, y, xn, yn

    pred_area, px, py, pxn, pyn = shoelace(pv, 0)
    gt_area, _, _, _, _ = shoelace(pv, 1)

    xmin = jnp.minimum(px, pxn) - 0.001
    xmax = jnp.maximum(px, pxn) + 0.001
    ymin = jnp.minimum(py, pyn) - 0.001
    ymax_e = jnp.maximum(py, pyn) + 0.001
    sign = jnp.where(pxn > px, 1.0, -1.0).astype(_F32)

    def unit_vec(sx, sy, ex, ey):
        vxr = ex - sx + 1e-6
        vyr = ey - sy + 1e-6
        nsq = vxr * vxr + vyr * vyr
        yi = 0x5F3759DF - (plsc.bitcast(nsq, jnp.int32) >> 1)
        ry = plsc.bitcast(yi, _F32)
        for _ in range(3):
            ry = ry * (1.5 - 0.5 * nsq * ry * ry)
        return vxr * ry, vyr * ry, nsq * ry

    fvx, fvy, fnorm = unit_vec(px, py, pxn, pyn)
    bvx, bvy, bnorm = unit_vec(pxn, pyn, px, py)

    def sample(dsf, sx, sy, vx, vy, flp):
        xs = sx + dsf * vx
        ys = sy + dsf * vy
        keep = ((xs <= xmax) & (xs >= xmin)
                & (ys <= ymax_e) & (ys >= ymin))
        fl = xs.astype(jnp.int32)
        uniq = keep & (fl != flp)
        y0i = ys.astype(jnp.int32)
        fx = xs - fl.astype(_F32)
        fy = ys - y0i.astype(_F32)
        y1i = y0i + 1
        k0 = (y0i >> 2) * _DIM + fl
        k1 = (y1i >> 2) * _DIM + fl
        sh0 = (y0i & 3) << 3
        sh1 = (y1i & 3) << 3

        def tap(k, sh):
            w = plsc.load_gather(maskv, [k], mask=uniq)
            return ((w >> sh) & 1).astype(_F32)

        m00 = tap(k0, sh0)
        m01 = tap(k1, sh1)
        m10 = tap(k0 + 1, sh0)
        m11 = tap(k1 + 1, sh1)
        a = m00 + fy * (m01 - m00)
        bb = m10 + fy * (m11 - m10)
        v = a + fx * (bb - a)
        return jnp.where(uniq, v, 0.0), fl

    def step(j, carry):
        acc_f, acc_b, flp_f, flp_b = carry
        dsf = (j * 2).astype(_F32)
        vf0, fl_f0 = sample(dsf, px, py, fvx, fvy, flp_f)
        vb0, fl_b0 = sample(dsf, pxn, pyn, bvx, bvy, flp_b)
        vf1, fl_f1 = sample(dsf + 1.0, px, py, fvx, fvy, fl_f0)
        vb1, fl_b1 = sample(dsf + 1.0, pxn, pyn, bvx, bvy, fl_b0)
        return acc_f + (vf0 + vf1), acc_b + (vb0 + vb1), fl_f1, fl_b1

    trip = jnp.minimum(
        jnp.max(jnp.maximum(fnorm, bnorm).astype(jnp.int32)) + 3, _NS)
    zerov = jnp.zeros_like(px)
    sentinel = jnp.zeros((_V,), jnp.int32) - (1 << 20)
    mask_cp.wait()
    acc_f, acc_b, _, _ = lax.fori_loop(
        0, (trip + 1) >> 1, step, (zerov, zerov, sentinel, sentinel))
    int_area = jnp.abs(jnp.sum(sign * (acc_f + acc_b) * 0.5))
    union = pred_area + gt_area - int_area
    zeros = jnp.zeros((_V,), _F32)
    outv[...] = (zeros + int_area) / (zeros + union)
    pltpu.sync_copy(outv, out_hbm.at[b])


@jax.jit
def kernel(poly, gt, gt_mask):
    def rows(p):
        x = p[:, :, 0]
        y = p[:, :, 1]
        return [x, y, jnp.roll(x, -1, axis=1), jnp.roll(y, -1, axis=1)]

    coords = jnp.concatenate(rows(poly) + rows(gt), axis=1)
    mi = gt_mask.astype(jnp.int32).reshape(_B, _DIM, _DIM)
    w = (mi[:, 0::4] | (mi[:, 1::4] << 8)
         | (mi[:, 2::4] << 16) | (mi[:, 3::4] << 24))
    maskf = jnp.pad(w.reshape(_B, (_DIM * _DIM) // 4), ((0, 0), (0, 60)))
    mesh = plsc.VectorSubcoreMesh(core_axis_name="c", subcore_axis_name="s")
    out = pl.kernel(
        _sc_body,
        mesh=mesh,
        compiler_params=pltpu.CompilerParams(
            needs_layout_passes=False, use_tc_tiling_on_sc=False),
        out_type=jax.ShapeDtypeStruct((_B, _V), _F32),
        scratch_types=[
            pltpu.VMEM((2560,), jnp.int32),
            pltpu.VMEM((8 * _V,), _F32),
            pltpu.VMEM((_V,), _F32),
            pltpu.SemaphoreType.DMA,
        ],
    )(coords, maskf)
    return out[:, 0]
